# baseline (speedup 1.0000x reference)
import functools

import jax
import jax.numpy as jnp
from jax import lax
from jax.experimental import pallas as pl
from jax.experimental.pallas import tpu as pltpu
from jax.experimental.pallas import tpu_sc as plsc

L = 16
NC, NS = 2, 16
NW = NC * NS
GR = 128
GB = 104


def _fm_call(x_t, emb_table, fc_flat, bias16, B, F, D, total):
    field_size = total // F
    rpw = B // NW
    nch = rpw // GR
    ppc = GR * F
    ng = ppc // L
    ngath = ppc // GB

    mesh = plsc.VectorSubcoreMesh(
        core_axis_name="c", subcore_axis_name="s", num_cores=NC, num_subcores=NS)

    @functools.partial(
        pl.kernel,
        out_type=jax.ShapeDtypeStruct((B,), jnp.float32),
        mesh=mesh,
        scratch_types=[
            pltpu.VMEM((ppc,), jnp.int32),
            pltpu.VMEM((ppc,), jnp.int32),
            pltpu.VMEM((ppc, D), jnp.float32),
            pltpu.VMEM((ppc,), jnp.float32),
            pltpu.VMEM((rpw,), jnp.float32),
            pltpu.VMEM((L,), jnp.float32),
            pltpu.SemaphoreType.DMA,
        ],
        compiler_params=pltpu.CompilerParams(
            needs_layout_passes=False, use_tc_tiling_on_sc=False),
    )
    def fm(emb_hbm, fc_hbm, x_hbm, b_hbm, out_hbm,
           xbuf, idxbuf, ebuf, fcbuf, obuf, bbuf, sem):
        w = lax.axis_index("s") * NC + lax.axis_index("c")
        pltpu.sync_copy(b_hbm, bbuf)
        iota = lax.iota(jnp.int32, L)
        m_last = iota == (L - 1)
        zero_i = iota * 0
        bias_v = bbuf[pl.ds(0, L)]

        def chunk_body(c, _):
            pltpu.sync_copy(x_hbm.at[pl.ds((w * nch + c) * ppc, ppc)], xbuf)

            def idx_body(g, _):
                off = pl.multiple_of(g * L, L)
                xv = xbuf[pl.ds(off, L)]
                idxbuf[pl.ds(off, L)] = xv + (g >> 3) * field_size
                return 0

            lax.fori_loop(0, ng, idx_body, 0)

            copies = [pltpu.async_copy(fc_hbm.at[idxbuf], fcbuf, sem)]
            for j in range(ngath):
                copies.append(pltpu.async_copy(
                    emb_hbm.at[idxbuf.at[pl.ds(j * GB, GB)]],
                    ebuf.at[pl.ds(j * GB, GB)], sem))
            for cp in copies:
                cp.wait()

            def row_body(i, _):
                for k in range(2):
                    rr = i * 2 + k
                    sa = [None] * 4
                    qa = [None] * 4
                    for f in range(F):
                        e = ebuf[f * GR + rr]
                        a = f % 4
                        sa[a] = e if sa[a] is None else sa[a] + e
                        qa[a] = e * e if qa[a] is None else qa[a] + e * e
                    s = (sa[0] + sa[1]) + (sa[2] + sa[3])
                    ss = (qa[0] + qa[1]) + (qa[2] + qa[3])
                    zc = jnp.cumsum(0.5 * (s * s - ss))
                    pos = zero_i + (c * GR + rr)
                    plsc.store_scatter(obuf, [pos], zc, mask=m_last)
                return 0

            lax.fori_loop(0, GR // 2, row_body, 0)

            def lin_body(i, _):
                rr = i * L
                lin = bias_v
                for f in range(F):
                    lin = lin + fcbuf[pl.ds(f * GR + rr, L)]
                z = obuf[pl.ds(c * GR + rr, L)] + lin
                obuf[pl.ds(c * GR + rr, L)] = 1.0 / (1.0 + jnp.exp(-z))
                return 0

            lax.fori_loop(0, GR // L, lin_body, 0)
            return 0

        lax.fori_loop(0, nch, chunk_body, 0)
        pltpu.sync_copy(obuf, out_hbm.at[pl.ds(w * rpw, rpw)])

    return fm(emb_table, fc_flat, x_t, bias16)


def kernel(x, emb_table, fc_table, bias):
    B, F = x.shape
    total, D = emb_table.shape
    assert D == L and B % (NW * GR) == 0 and total % F == 0
    x_t = jnp.transpose(
        x.astype(jnp.int32).reshape(B // GR, GR, F), (0, 2, 1)).reshape(-1)
    fc_flat = fc_table.reshape(-1)
    bias16 = jnp.broadcast_to(bias.astype(jnp.float32), (L,))
    return _fm_call(x_t, emb_table, fc_flat, bias16, B, F, D, total)


# speedup vs baseline: 1.0796x; 1.0796x over previous
import functools

import jax
import jax.numpy as jnp
from jax import lax
from jax.experimental import pallas as pl
from jax.experimental.pallas import tpu as pltpu
from jax.experimental.pallas import tpu_sc as plsc

L = 16
NC, NS = 2, 16
NW = NC * NS
GR = 128
GB = 104


def _fm_call(x_t, emb_cols, fc_flat, bias16, B, F, D, total):
    field_size = total // F
    rpw = B // NW
    nch = rpw // GR
    ppc = GR * F
    ng = ppc // L
    ngath = ppc // GB

    mesh = plsc.VectorSubcoreMesh(
        core_axis_name="c", subcore_axis_name="s", num_cores=NC, num_subcores=NS)

    @functools.partial(
        pl.kernel,
        out_type=jax.ShapeDtypeStruct((B,), jnp.float32),
        mesh=mesh,
        scratch_types=[
            pltpu.VMEM((ppc,), jnp.int32),
            pltpu.VMEM((ppc,), jnp.int32),
            *[pltpu.VMEM((ppc,), jnp.float32) for _ in range(D)],
            pltpu.VMEM((ppc,), jnp.float32),
            pltpu.VMEM((rpw,), jnp.float32),
            pltpu.VMEM((L,), jnp.float32),
            pltpu.SemaphoreType.DMA,
        ],
        compiler_params=pltpu.CompilerParams(
            needs_layout_passes=False, use_tc_tiling_on_sc=False),
    )
    def fm(*args):
        cols_hbm = args[:D]
        fc_hbm, x_hbm, b_hbm, out_hbm, xbuf, idxbuf = args[D:D + 6]
        ebufs = args[D + 6:2 * D + 6]
        fcbuf, obuf, bbuf, sem = args[2 * D + 6:]
        w = lax.axis_index("s") * NC + lax.axis_index("c")
        pltpu.sync_copy(b_hbm, bbuf)
        iota = lax.iota(jnp.int32, L)
        m_last = iota == (L - 1)
        zero_i = iota * 0
        bias_v = bbuf[pl.ds(0, L)]

        def chunk_body(c, _):
            pltpu.sync_copy(x_hbm.at[pl.ds((w * nch + c) * ppc, ppc)], xbuf)

            def idx_body(g, _):
                off = pl.multiple_of(g * L, L)
                xv = xbuf[pl.ds(off, L)]
                idxbuf[pl.ds(off, L)] = xv + (g >> 3) * field_size
                return 0

            lax.fori_loop(0, ng, idx_body, 0)

            copies = [pltpu.async_copy(fc_hbm.at[idxbuf], fcbuf, sem)]
            for d in range(D):
                copies.append(pltpu.async_copy(
                    cols_hbm[d].at[idxbuf], ebufs[d], sem))
            for cp in copies:
                cp.wait()

            def rows_body(i, _):
                rr = i * L
                lin = bias_v
                for f in range(F):
                    lin = lin + fcbuf[pl.ds(f * GR + rr, L)]
                zacc = lin
                for d in range(D):
                    s = None
                    ss = None
                    for f in range(F):
                        e = ebufs[d][pl.ds(f * GR + rr, L)]
                        s = e if s is None else s + e
                        ss = e * e if ss is None else ss + e * e
                    zacc = zacc + 0.5 * (s * s - ss)
                obuf[pl.ds(c * GR + rr, L)] = 1.0 / (1.0 + jnp.exp(-zacc))
                return 0

            lax.fori_loop(0, GR // L, rows_body, 0)
            return 0

        lax.fori_loop(0, nch, chunk_body, 0)
        pltpu.sync_copy(obuf, out_hbm.at[pl.ds(w * rpw, rpw)])

    return fm(*emb_cols, fc_flat, x_t, bias16)


def kernel(x, emb_table, fc_table, bias):
    B, F = x.shape
    total, D = emb_table.shape
    assert D == L and B % (NW * GR) == 0 and total % F == 0
    x_t = jnp.transpose(
        x.astype(jnp.int32).reshape(B // GR, GR, F), (0, 2, 1)).reshape(-1)
    emb_cols = [emb_table[:, d] for d in range(D)]
    fc_flat = fc_table.reshape(-1)
    bias16 = jnp.broadcast_to(bias.astype(jnp.float32), (L,))
    return _fm_call(x_t, emb_cols, fc_flat, bias16, B, F, D, total)
